# Initial kernel scaffold; baseline (speedup 1.0000x reference)
import jax
import jax.numpy as jnp
from jax.experimental import pallas as pl


def kernel(x, W1_a, Wmu_a, Wls_a, W1_b, Wmu_b, Wls_b, W_out, b_out, edge_index):
    raise NotImplementedError("write your pallas kernel here")


# speedup vs baseline: 11.2330x; 11.2330x over previous
import functools

import jax
import jax.numpy as jnp
from jax import lax
from jax.experimental import pallas as pl
from jax.experimental.pallas import tpu as pltpu
from jax.experimental.pallas import tpu_sc as plsc

N = 10000
E = 320000
DF = 128
HID = 64
LAT = 32
OUTD = 64
MAXLS = 10.0

NC, NS = 2, 16
NW = NC * NS
EPW = 10240
E_PAD = NW * EPW
CB = 128
NCH = EPW // CB
RPT = 640
R_ACC = NS * RPT
PAD_DST = 10008
DEG_ACC = 10240

BN = 1024
GRID = (N + BN - 1) // BN


def _sc_mesh():
    return plsc.VectorSubcoreMesh(core_axis_name="c", subcore_axis_name="s")


@functools.partial(
    pl.kernel,
    mesh=_sc_mesh(),
    out_type=jax.ShapeDtypeStruct((NW, DEG_ACC), jnp.float32),
    scratch_types=[
        pltpu.VMEM((EPW,), jnp.int32),
        pltpu.VMEM((DEG_ACC,), jnp.float32),
    ],
    compiler_params=pltpu.CompilerParams(needs_layout_passes=False),
)
def _sc_degree(dst_hbm, out_hbm, dst_v, acc_v):
    cid = lax.axis_index("c")
    sid = lax.axis_index("s")
    wid = cid * NS + sid
    pltpu.sync_copy(dst_hbm.at[wid], dst_v)

    def zero(i, _):
        acc_v[pl.ds(i * 16, 16)] = jnp.zeros((16,), jnp.float32)
        return 0

    lax.fori_loop(0, DEG_ACC // 16, zero, 0)

    ones = jnp.ones((16,), jnp.float32)

    def body(i, _):
        idx = dst_v[pl.ds(i * 16, 16)]
        plsc.addupdate_scatter(acc_v, [idx], ones)
        return 0

    lax.fori_loop(0, EPW // 16, body, 0)
    pltpu.sync_copy(acc_v, out_hbm.at[wid])


@functools.partial(
    pl.kernel,
    mesh=_sc_mesh(),
    out_type=jax.ShapeDtypeStruct((NC, R_ACC, DF), jnp.float32),
    scratch_types=[
        pltpu.VMEM((NCH, CB), jnp.int32),
        pltpu.VMEM((NCH, CB), jnp.int32),
        pltpu.VMEM((CB, DF), jnp.float32),
        pltpu.VMEM_SHARED((R_ACC, DF), jnp.float32),
        pltpu.SemaphoreType.DMA,
    ],
)
def _sc_agg(m_hbm, src_hbm, dst_hbm, z_hbm, out_hbm,
            src_v, dst_v, rows_v, acc_s, sem):
    cid = lax.axis_index("c")
    sid = lax.axis_index("s")
    wid = cid * NS + sid
    pltpu.sync_copy(z_hbm, acc_s.at[pl.ds(sid * RPT, RPT)])
    pltpu.sync_copy(src_hbm.at[wid], src_v)
    pltpu.sync_copy(dst_hbm.at[wid], dst_v)
    plsc.subcore_barrier()

    def chunk(j, _):
        pltpu.async_copy(m_hbm.at[src_v.at[j]], rows_v, sem).wait()
        pltpu.sync_copy(rows_v, acc_s.at[dst_v.at[j]], add=True)
        return 0

    lax.fori_loop(0, NCH, chunk, 0)
    plsc.subcore_barrier()
    pltpu.sync_copy(acc_s.at[pl.ds(sid * RPT, RPT)],
                    out_hbm.at[cid, pl.ds(sid * RPT, RPT)])


def _dense1_body(degp, x, w1, m1_o, dsc_o):
    deg = jnp.maximum(jnp.sum(degp[...], axis=0), 1.0)
    dsc = lax.rsqrt(deg)
    m1_o[...] = jnp.dot(x[...], w1[...],
                        preferred_element_type=jnp.float32) * dsc[:, None]
    dsc_o[...] = dsc


def _dense2_body(aggp, dsc, w2, m2_o):
    a = (aggp[0] + aggp[1]) * dsc[...][:, None]
    h = jnp.maximum(a, 0.0)
    m2_o[...] = jnp.dot(h, w2[...],
                        preferred_element_type=jnp.float32) * dsc[...][:, None]


def _dense3_body(aggp, dsc, eps, wo, bo, z_o):
    t = (aggp[0] + aggp[1]) * dsc[...][:, None]
    mu = jnp.concatenate([t[:, 0:LAT], t[:, 2 * LAT:3 * LAT]], axis=1)
    ls = jnp.concatenate([t[:, LAT:2 * LAT], t[:, 3 * LAT:4 * LAT]], axis=1)
    z = mu + eps[...] * jnp.exp(jnp.minimum(ls, MAXLS))
    z_o[...] = jnp.dot(z, wo[...],
                       preferred_element_type=jnp.float32) + bo[...]


_dense1 = pl.pallas_call(
    _dense1_body,
    grid=(GRID,),
    in_specs=[
        pl.BlockSpec((NW, BN), lambda i: (0, i)),
        pl.BlockSpec((BN, DF), lambda i: (i, 0)),
        pl.BlockSpec((DF, DF), lambda i: (0, 0)),
    ],
    out_specs=[
        pl.BlockSpec((BN, DF), lambda i: (i, 0)),
        pl.BlockSpec((BN,), lambda i: (i,)),
    ],
    out_shape=[
        jax.ShapeDtypeStruct((N, DF), jnp.float32),
        jax.ShapeDtypeStruct((N,), jnp.float32),
    ],
)

_dense2 = pl.pallas_call(
    _dense2_body,
    grid=(GRID,),
    in_specs=[
        pl.BlockSpec((NC, BN, DF), lambda i: (0, i, 0)),
        pl.BlockSpec((BN,), lambda i: (i,)),
        pl.BlockSpec((DF, DF), lambda i: (0, 0)),
    ],
    out_specs=pl.BlockSpec((BN, DF), lambda i: (i, 0)),
    out_shape=jax.ShapeDtypeStruct((N, DF), jnp.float32),
)

_dense3 = pl.pallas_call(
    _dense3_body,
    grid=(GRID,),
    in_specs=[
        pl.BlockSpec((NC, BN, DF), lambda i: (0, i, 0)),
        pl.BlockSpec((BN,), lambda i: (i,)),
        pl.BlockSpec((BN, 2 * LAT), lambda i: (i, 0)),
        pl.BlockSpec((2 * LAT, OUTD), lambda i: (0, 0)),
        pl.BlockSpec((1, OUTD), lambda i: (0, 0)),
    ],
    out_specs=pl.BlockSpec((BN, OUTD), lambda i: (i, 0)),
    out_shape=jax.ShapeDtypeStruct((N, OUTD), jnp.float32),
)


def kernel(x, W1_a, Wmu_a, Wls_a, W1_b, Wmu_b, Wls_b, W_out, b_out,
           edge_index):
    f32 = jnp.float32
    W1c = jnp.concatenate([W1_a, W1_b], axis=1)
    zblk = jnp.zeros((HID, 2 * LAT), f32)
    W2 = jnp.concatenate([
        jnp.concatenate([Wmu_a, Wls_a, zblk], axis=1),
        jnp.concatenate([zblk, Wmu_b, Wls_b], axis=1),
    ], axis=0)
    ke_a, ke_b = jax.random.split(jax.random.key(42), 2)
    eps = jnp.concatenate([
        jax.random.normal(ke_a, (N, LAT), dtype=f32),
        jax.random.normal(ke_b, (N, LAT), dtype=f32),
    ], axis=1)

    pad_n = E_PAD - E
    src_p = jnp.concatenate([edge_index[0],
                             jnp.zeros((pad_n,), jnp.int32)])
    dst_p = jnp.concatenate([edge_index[1],
                             jnp.full((pad_n,), PAD_DST, jnp.int32)])
    src_r = src_p.reshape(NW, NCH, CB)
    dst_r = dst_p.reshape(NW, NCH, CB)
    dst_f = dst_p.reshape(NW, EPW)
    zrows = jnp.zeros((RPT, DF), f32)

    degp = _sc_degree(dst_f)
    m1, dsc = _dense1(degp, x, W1c)
    agg1 = _sc_agg(m1, src_r, dst_r, zrows)
    m2 = _dense2(agg1, dsc, W2)
    agg2 = _sc_agg(m2, src_r, dst_r, zrows)
    z = _dense3(agg2, dsc, eps, W_out, b_out.reshape(1, OUTD))
    return z


# speedup vs baseline: 12.6754x; 1.1284x over previous
import functools

import jax
import jax.numpy as jnp
from jax import lax
from jax.experimental import pallas as pl
from jax.experimental.pallas import tpu as pltpu
from jax.experimental.pallas import tpu_sc as plsc

N = 10000
E = 320000
DF = 128
HF = 64
HID = 64
LAT = 32
OUTD = 64
MAXLS = 10.0

NC, NS = 2, 16
NW = NC * NS
EPW = 10240
E_PAD = NW * EPW
CB = 128
NCH = EPW // CB
RPT = 640
R_ACC = NS * RPT
PAD_DST = 10008
DEG_ACC = 10240

BN = 1024
GRID = (N + BN - 1) // BN


def _sc_mesh():
    return plsc.VectorSubcoreMesh(core_axis_name="c", subcore_axis_name="s")


@functools.partial(
    pl.kernel,
    mesh=_sc_mesh(),
    out_type=jax.ShapeDtypeStruct((NW, DEG_ACC), jnp.float32),
    scratch_types=[
        pltpu.VMEM((EPW,), jnp.int32),
        pltpu.VMEM((DEG_ACC,), jnp.float32),
    ],
    compiler_params=pltpu.CompilerParams(needs_layout_passes=False),
)
def _sc_degree(dst_hbm, out_hbm, dst_v, acc_v):
    cid = lax.axis_index("c")
    sid = lax.axis_index("s")
    wid = cid * NS + sid
    pltpu.sync_copy(dst_hbm.at[wid], dst_v)

    def zero(i, _):
        acc_v[pl.ds(i * 16, 16)] = jnp.zeros((16,), jnp.float32)
        return 0

    lax.fori_loop(0, DEG_ACC // 16, zero, 0)

    ones = jnp.ones((16,), jnp.float32)

    def body(i, _):
        idx = dst_v[pl.ds(i * 16, 16)]
        plsc.addupdate_scatter(acc_v, [idx], ones)
        return 0

    lax.fori_loop(0, EPW // 16, body, 0)
    pltpu.sync_copy(acc_v, out_hbm.at[wid])


@functools.partial(
    pl.kernel,
    mesh=_sc_mesh(),
    out_type=[
        jax.ShapeDtypeStruct((NC, R_ACC, HF), jnp.float32),
        jax.ShapeDtypeStruct((NC, R_ACC, HF), jnp.float32),
    ],
    scratch_types=[
        pltpu.VMEM((NCH, CB), jnp.int32),
        pltpu.VMEM((NCH, CB), jnp.int32),
        pltpu.VMEM((3, CB, HF), jnp.float32),
        pltpu.VMEM_SHARED((R_ACC, HF), jnp.float32),
        pltpu.SemaphoreType.DMA((3,)),
        pltpu.SemaphoreType.DMA((3,)),
    ],
    compiler_params=pltpu.CompilerParams(use_tc_tiling_on_sc=False),
)
def _sc_agg(m_lo, m_hi, src_hbm, dst_hbm, z_hbm, o_lo, o_hi,
            src_v, dst_v, rows_v, acc_s, gsem, ssem):
    cid = lax.axis_index("c")
    sid = lax.axis_index("s")
    wid = cid * NS + sid
    pltpu.sync_copy(src_hbm.at[wid], src_v)
    pltpu.sync_copy(dst_hbm.at[wid], dst_v)

    for m_hbm, out_hbm in ((m_lo, o_lo), (m_hi, o_hi)):
        pltpu.sync_copy(z_hbm, acc_s.at[pl.ds(sid * RPT, RPT)])
        plsc.subcore_barrier()

        def gather(j, m_hbm=m_hbm):
            return pltpu.async_copy(m_hbm.at[src_v.at[j]], rows_v.at[j % 3],
                                    gsem.at[j % 3])

        def scatter(j):
            return pltpu.async_copy(rows_v.at[j % 3], acc_s.at[dst_v.at[j]],
                                    ssem.at[j % 3], add=True)

        g_h = {0: gather(0), 1: gather(1)}
        s_h = {}
        for j in range(NCH):
            g_h.pop(j).wait()
            if j - 1 in s_h:
                s_h.pop(j - 1).wait()
            if j + 2 < NCH:
                g_h[j + 2] = gather(j + 2)
            s_h[j] = scatter(j)
        s_h.pop(NCH - 1).wait()
        plsc.subcore_barrier()
        pltpu.sync_copy(acc_s.at[pl.ds(sid * RPT, RPT)],
                        out_hbm.at[cid, pl.ds(sid * RPT, RPT)])
        plsc.subcore_barrier()


def _dense1_body(degp, x, w1, lo_o, hi_o, dsc_o):
    deg = jnp.maximum(jnp.sum(degp[...], axis=0), 1.0)
    dsc = lax.rsqrt(deg)
    m = jnp.dot(x[...], w1[...],
                preferred_element_type=jnp.float32) * dsc[:, None]
    lo_o[...] = m[:, :HF]
    hi_o[...] = m[:, HF:]
    dsc_o[...] = dsc


def _dense2_body(alo, ahi, dsc, w2, lo_o, hi_o):
    a = jnp.concatenate([alo[0] + alo[1], ahi[0] + ahi[1]], axis=1)
    h = jnp.maximum(a * dsc[...][:, None], 0.0)
    m = jnp.dot(h, w2[...],
                preferred_element_type=jnp.float32) * dsc[...][:, None]
    lo_o[...] = m[:, :HF]
    hi_o[...] = m[:, HF:]


def _dense3_body(alo, ahi, dsc, eps, wo, bo, z_o):
    t = jnp.concatenate([alo[0] + alo[1], ahi[0] + ahi[1]],
                        axis=1) * dsc[...][:, None]
    mu = jnp.concatenate([t[:, 0:LAT], t[:, 2 * LAT:3 * LAT]], axis=1)
    ls = jnp.concatenate([t[:, LAT:2 * LAT], t[:, 3 * LAT:4 * LAT]], axis=1)
    z = mu + eps[...] * jnp.exp(jnp.minimum(ls, MAXLS))
    z_o[...] = jnp.dot(z, wo[...],
                       preferred_element_type=jnp.float32) + bo[...]


_dense1 = pl.pallas_call(
    _dense1_body,
    grid=(GRID,),
    in_specs=[
        pl.BlockSpec((NW, BN), lambda i: (0, i)),
        pl.BlockSpec((BN, DF), lambda i: (i, 0)),
        pl.BlockSpec((DF, DF), lambda i: (0, 0)),
    ],
    out_specs=[
        pl.BlockSpec((BN, HF), lambda i: (i, 0)),
        pl.BlockSpec((BN, HF), lambda i: (i, 0)),
        pl.BlockSpec((BN,), lambda i: (i,)),
    ],
    out_shape=[
        jax.ShapeDtypeStruct((N, HF), jnp.float32),
        jax.ShapeDtypeStruct((N, HF), jnp.float32),
        jax.ShapeDtypeStruct((N,), jnp.float32),
    ],
)

_agg_spec = pl.BlockSpec((NC, BN, HF), lambda i: (0, i, 0))

_dense2 = pl.pallas_call(
    _dense2_body,
    grid=(GRID,),
    in_specs=[
        _agg_spec,
        _agg_spec,
        pl.BlockSpec((BN,), lambda i: (i,)),
        pl.BlockSpec((DF, DF), lambda i: (0, 0)),
    ],
    out_specs=[
        pl.BlockSpec((BN, HF), lambda i: (i, 0)),
        pl.BlockSpec((BN, HF), lambda i: (i, 0)),
    ],
    out_shape=[
        jax.ShapeDtypeStruct((N, HF), jnp.float32),
        jax.ShapeDtypeStruct((N, HF), jnp.float32),
    ],
)

_dense3 = pl.pallas_call(
    _dense3_body,
    grid=(GRID,),
    in_specs=[
        _agg_spec,
        _agg_spec,
        pl.BlockSpec((BN,), lambda i: (i,)),
        pl.BlockSpec((BN, 2 * LAT), lambda i: (i, 0)),
        pl.BlockSpec((2 * LAT, OUTD), lambda i: (0, 0)),
        pl.BlockSpec((1, OUTD), lambda i: (0, 0)),
    ],
    out_specs=pl.BlockSpec((BN, OUTD), lambda i: (i, 0)),
    out_shape=jax.ShapeDtypeStruct((N, OUTD), jnp.float32),
)


def kernel(x, W1_a, Wmu_a, Wls_a, W1_b, Wmu_b, Wls_b, W_out, b_out,
           edge_index):
    f32 = jnp.float32
    W1c = jnp.concatenate([W1_a, W1_b], axis=1)
    zblk = jnp.zeros((HID, 2 * LAT), f32)
    W2 = jnp.concatenate([
        jnp.concatenate([Wmu_a, Wls_a, zblk], axis=1),
        jnp.concatenate([zblk, Wmu_b, Wls_b], axis=1),
    ], axis=0)
    ke_a, ke_b = jax.random.split(jax.random.key(42), 2)
    eps = jnp.concatenate([
        jax.random.normal(ke_a, (N, LAT), dtype=f32),
        jax.random.normal(ke_b, (N, LAT), dtype=f32),
    ], axis=1)

    pad_n = E_PAD - E
    src_p = jnp.concatenate([edge_index[0],
                             jnp.zeros((pad_n,), jnp.int32)])
    dst_p = jnp.concatenate([edge_index[1],
                             jnp.full((pad_n,), PAD_DST, jnp.int32)])
    src_r = src_p.reshape(NW, NCH, CB)
    dst_r = dst_p.reshape(NW, NCH, CB)
    dst_f = dst_p.reshape(NW, EPW)
    zrows = jnp.zeros((RPT, HF), f32)

    degp = _sc_degree(dst_f)
    m1lo, m1hi, dsc = _dense1(degp, x, W1c)
    a1lo, a1hi = _sc_agg(m1lo, m1hi, src_r, dst_r, zrows)
    m2lo, m2hi = _dense2(a1lo, a1hi, dsc, W2)
    a2lo, a2hi = _sc_agg(m2lo, m2hi, src_r, dst_r, zrows)
    z = _dense3(a2lo, a2hi, dsc, eps, W_out, b_out.reshape(1, OUTD))
    return z


# speedup vs baseline: 13.0089x; 1.0263x over previous
import functools

import jax
import jax.numpy as jnp
from jax import lax
from jax.experimental import pallas as pl
from jax.experimental.pallas import tpu as pltpu
from jax.experimental.pallas import tpu_sc as plsc

N = 10000
E = 320000
DF = 128
HF = 64
HID = 64
LAT = 32
OUTD = 64
MAXLS = 10.0

NC, NS = 2, 16
NW = NC * NS
EPW = 10240
E_PAD = NW * EPW
CB = 128
NCH = EPW // CB
RPT = 640
R_ACC = NS * RPT
PAD_DST = 10008
DEG_ACC = 10240

BN = 1024
GRID = (N + BN - 1) // BN


def _sc_mesh():
    return plsc.VectorSubcoreMesh(core_axis_name="c", subcore_axis_name="s")


@functools.partial(
    pl.kernel,
    mesh=_sc_mesh(),
    out_type=jax.ShapeDtypeStruct((NW, DEG_ACC), jnp.float32),
    scratch_types=[
        pltpu.VMEM((EPW,), jnp.int32),
        pltpu.VMEM((DEG_ACC,), jnp.float32),
    ],
    compiler_params=pltpu.CompilerParams(needs_layout_passes=False),
)
def _sc_degree(dst_hbm, out_hbm, dst_v, acc_v):
    cid = lax.axis_index("c")
    sid = lax.axis_index("s")
    wid = cid * NS + sid
    pltpu.sync_copy(dst_hbm.at[wid], dst_v)

    def zero(i, _):
        acc_v[pl.ds(i * 16, 16)] = jnp.zeros((16,), jnp.float32)
        return 0

    lax.fori_loop(0, DEG_ACC // 16, zero, 0)

    ones = jnp.ones((16,), jnp.float32)

    def body(i, _):
        idx = dst_v[pl.ds(i * 16, 16)]
        plsc.addupdate_scatter(acc_v, [idx], ones)
        return 0

    lax.fori_loop(0, EPW // 16, body, 0)
    pltpu.sync_copy(acc_v, out_hbm.at[wid])


@functools.partial(
    pl.kernel,
    mesh=_sc_mesh(),
    out_type=[
        jax.ShapeDtypeStruct((NC, R_ACC, HF), jnp.float32),
        jax.ShapeDtypeStruct((NC, R_ACC, HF), jnp.float32),
    ],
    scratch_types=[
        pltpu.VMEM((NCH, CB), jnp.int32),
        pltpu.VMEM((NCH, CB), jnp.int32),
        pltpu.VMEM((4, CB, HF), jnp.float32),
        pltpu.VMEM_SHARED((R_ACC, HF), jnp.float32),
        pltpu.SemaphoreType.DMA((4,)),
        pltpu.SemaphoreType.DMA((4,)),
    ],
    compiler_params=pltpu.CompilerParams(use_tc_tiling_on_sc=False),
)
def _sc_agg(m_lo, m_hi, src_hbm, dst_hbm, z_hbm, o_lo, o_hi,
            src_v, dst_v, rows_v, acc_s, gsem, ssem):
    cid = lax.axis_index("c")
    sid = lax.axis_index("s")
    wid = cid * NS + sid
    pltpu.sync_copy(src_hbm.at[wid], src_v)
    pltpu.sync_copy(dst_hbm.at[wid], dst_v)

    for m_hbm, out_hbm in ((m_lo, o_lo), (m_hi, o_hi)):
        pltpu.sync_copy(z_hbm, acc_s.at[pl.ds(sid * RPT, RPT)])
        plsc.subcore_barrier()

        def gather(j, m_hbm=m_hbm):
            return pltpu.async_copy(m_hbm.at[src_v.at[j]], rows_v.at[j % 4],
                                    gsem.at[j % 4])

        def scatter(j):
            return pltpu.async_copy(rows_v.at[j % 4], acc_s.at[dst_v.at[j]],
                                    ssem.at[j % 4], add=True)

        g_h = {0: gather(0), 1: gather(1)}
        s_h = {}
        for j in range(NCH):
            g_h.pop(j).wait()
            if j - 2 in s_h:
                s_h.pop(j - 2).wait()
            if j + 2 < NCH:
                g_h[j + 2] = gather(j + 2)
            s_h[j] = scatter(j)
        for j in sorted(s_h):
            s_h[j].wait()
        plsc.subcore_barrier()
        pltpu.sync_copy(acc_s.at[pl.ds(sid * RPT, RPT)],
                        out_hbm.at[cid, pl.ds(sid * RPT, RPT)])
        plsc.subcore_barrier()


def _dense1_body(degp, x, w1, lo_o, hi_o, dsc_o):
    deg = jnp.maximum(jnp.sum(degp[...], axis=0), 1.0)
    dsc = lax.rsqrt(deg)
    m = jnp.dot(x[...], w1[...],
                preferred_element_type=jnp.float32) * dsc[:, None]
    lo_o[...] = m[:, :HF]
    hi_o[...] = m[:, HF:]
    dsc_o[...] = dsc


def _dense2_body(alo, ahi, dsc, w2, lo_o, hi_o):
    a = jnp.concatenate([alo[0] + alo[1], ahi[0] + ahi[1]], axis=1)
    h = jnp.maximum(a * dsc[...][:, None], 0.0)
    m = jnp.dot(h, w2[...],
                preferred_element_type=jnp.float32) * dsc[...][:, None]
    lo_o[...] = m[:, :HF]
    hi_o[...] = m[:, HF:]


def _dense3_body(alo, ahi, dsc, eps, wo, bo, z_o):
    t = jnp.concatenate([alo[0] + alo[1], ahi[0] + ahi[1]],
                        axis=1) * dsc[...][:, None]
    mu = jnp.concatenate([t[:, 0:LAT], t[:, 2 * LAT:3 * LAT]], axis=1)
    ls = jnp.concatenate([t[:, LAT:2 * LAT], t[:, 3 * LAT:4 * LAT]], axis=1)
    z = mu + eps[...] * jnp.exp(jnp.minimum(ls, MAXLS))
    z_o[...] = jnp.dot(z, wo[...],
                       preferred_element_type=jnp.float32) + bo[...]


_dense1 = pl.pallas_call(
    _dense1_body,
    grid=(GRID,),
    in_specs=[
        pl.BlockSpec((NW, BN), lambda i: (0, i)),
        pl.BlockSpec((BN, DF), lambda i: (i, 0)),
        pl.BlockSpec((DF, DF), lambda i: (0, 0)),
    ],
    out_specs=[
        pl.BlockSpec((BN, HF), lambda i: (i, 0)),
        pl.BlockSpec((BN, HF), lambda i: (i, 0)),
        pl.BlockSpec((BN,), lambda i: (i,)),
    ],
    out_shape=[
        jax.ShapeDtypeStruct((N, HF), jnp.float32),
        jax.ShapeDtypeStruct((N, HF), jnp.float32),
        jax.ShapeDtypeStruct((N,), jnp.float32),
    ],
)

_agg_spec = pl.BlockSpec((NC, BN, HF), lambda i: (0, i, 0))

_dense2 = pl.pallas_call(
    _dense2_body,
    grid=(GRID,),
    in_specs=[
        _agg_spec,
        _agg_spec,
        pl.BlockSpec((BN,), lambda i: (i,)),
        pl.BlockSpec((DF, DF), lambda i: (0, 0)),
    ],
    out_specs=[
        pl.BlockSpec((BN, HF), lambda i: (i, 0)),
        pl.BlockSpec((BN, HF), lambda i: (i, 0)),
    ],
    out_shape=[
        jax.ShapeDtypeStruct((N, HF), jnp.float32),
        jax.ShapeDtypeStruct((N, HF), jnp.float32),
    ],
)

_dense3 = pl.pallas_call(
    _dense3_body,
    grid=(GRID,),
    in_specs=[
        _agg_spec,
        _agg_spec,
        pl.BlockSpec((BN,), lambda i: (i,)),
        pl.BlockSpec((BN, 2 * LAT), lambda i: (i, 0)),
        pl.BlockSpec((2 * LAT, OUTD), lambda i: (0, 0)),
        pl.BlockSpec((1, OUTD), lambda i: (0, 0)),
    ],
    out_specs=pl.BlockSpec((BN, OUTD), lambda i: (i, 0)),
    out_shape=jax.ShapeDtypeStruct((N, OUTD), jnp.float32),
)


def kernel(x, W1_a, Wmu_a, Wls_a, W1_b, Wmu_b, Wls_b, W_out, b_out,
           edge_index):
    f32 = jnp.float32
    W1c = jnp.concatenate([W1_a, W1_b], axis=1)
    zblk = jnp.zeros((HID, 2 * LAT), f32)
    W2 = jnp.concatenate([
        jnp.concatenate([Wmu_a, Wls_a, zblk], axis=1),
        jnp.concatenate([zblk, Wmu_b, Wls_b], axis=1),
    ], axis=0)
    ke_a, ke_b = jax.random.split(jax.random.key(42), 2)
    eps = jnp.concatenate([
        jax.random.normal(ke_a, (N, LAT), dtype=f32),
        jax.random.normal(ke_b, (N, LAT), dtype=f32),
    ], axis=1)

    pad_n = E_PAD - E
    src_p = jnp.concatenate([edge_index[0],
                             jnp.zeros((pad_n,), jnp.int32)])
    dst_p = jnp.concatenate([edge_index[1],
                             jnp.full((pad_n,), PAD_DST, jnp.int32)])
    src_r = src_p.reshape(NW, NCH, CB)
    dst_r = dst_p.reshape(NW, NCH, CB)
    dst_f = dst_p.reshape(NW, EPW)
    zrows = jnp.zeros((RPT, HF), f32)

    degp = _sc_degree(dst_f)
    m1lo, m1hi, dsc = _dense1(degp, x, W1c)
    a1lo, a1hi = _sc_agg(m1lo, m1hi, src_r, dst_r, zrows)
    m2lo, m2hi = _dense2(a1lo, a1hi, dsc, W2)
    a2lo, a2hi = _sc_agg(m2lo, m2hi, src_r, dst_r, zrows)
    z = _dense3(a2lo, a2hi, dsc, eps, W_out, b_out.reshape(1, OUTD))
    return z


# speedup vs baseline: 22.6677x; 1.7425x over previous
import functools

import jax
import jax.numpy as jnp
from jax import lax
from jax.experimental import pallas as pl
from jax.experimental.pallas import tpu as pltpu
from jax.experimental.pallas import tpu_sc as plsc

N = 10000
E = 320000
DF = 128
HF = 64
HID = 64
LAT = 32
OUTD = 64
MAXLS = 10.0

NC, NS = 2, 16
NW = NC * NS
EPW = 10240
E_PAD = NW * EPW
CB = 128
NCH = EPW // CB
RPT = 640
R_ACC = NS * RPT
PAD_DST = 10008
DEG_ACC = 10240

BN = 1024
GRID = (N + BN - 1) // BN


def _sc_mesh():
    return plsc.VectorSubcoreMesh(core_axis_name="c", subcore_axis_name="s")


@functools.partial(
    pl.kernel,
    mesh=_sc_mesh(),
    out_type=jax.ShapeDtypeStruct((NW, DEG_ACC), jnp.float32),
    scratch_types=[
        pltpu.VMEM((EPW,), jnp.int32),
        pltpu.VMEM((DEG_ACC,), jnp.float32),
    ],
    compiler_params=pltpu.CompilerParams(needs_layout_passes=False),
)
def _sc_degree(dst_hbm, out_hbm, dst_v, acc_v):
    cid = lax.axis_index("c")
    sid = lax.axis_index("s")
    wid = cid * NS + sid
    pltpu.sync_copy(dst_hbm.at[wid], dst_v)

    def zero(i, _):
        acc_v[pl.ds(i * 16, 16)] = jnp.zeros((16,), jnp.float32)
        return 0

    lax.fori_loop(0, DEG_ACC // 16, zero, 0)

    ones = jnp.ones((16,), jnp.float32)

    def body(i, _):
        idx = dst_v[pl.ds(i * 16, 16)]
        plsc.addupdate_scatter(acc_v, [idx], ones)
        return 0

    lax.fori_loop(0, EPW // 16, body, 0)
    pltpu.sync_copy(acc_v, out_hbm.at[wid])


FAST_CID = 0
FAST_NCH = 121
SLOW_NCH = 36
E16 = E // 16


@functools.partial(
    pl.kernel,
    mesh=_sc_mesh(),
    out_type=[
        jax.ShapeDtypeStruct((NC, R_ACC, HF), jnp.float32),
        jax.ShapeDtypeStruct((NC, R_ACC, HF), jnp.float32),
    ],
    scratch_types=[
        pltpu.VMEM((FAST_NCH, CB), jnp.int32),
        pltpu.VMEM((FAST_NCH, CB), jnp.int32),
        pltpu.VMEM((4, CB, HF), jnp.float32),
        pltpu.VMEM_SHARED((R_ACC, HF), jnp.float32),
        pltpu.SemaphoreType.DMA((4,)),
        pltpu.SemaphoreType.DMA((4,)),
    ],
    compiler_params=pltpu.CompilerParams(use_tc_tiling_on_sc=False),
)
def _sc_agg(m_lo, m_hi, srcF, dstF, srcS, dstS, o_lo, o_hi,
            src_v, dst_v, rows_v, acc_s, gsem, ssem):
    cid = lax.axis_index("c")
    sid = lax.axis_index("s")
    is_fast = cid == FAST_CID
    nch = jnp.where(is_fast, FAST_NCH, SLOW_NCH)

    @pl.when(is_fast)
    def _():
        pltpu.sync_copy(srcF.at[sid], src_v)
        pltpu.sync_copy(dstF.at[sid], dst_v)

    @pl.when(jnp.logical_not(is_fast))
    def _():
        pltpu.sync_copy(srcS.at[sid], src_v.at[pl.ds(0, SLOW_NCH)])
        pltpu.sync_copy(dstS.at[sid], dst_v.at[pl.ds(0, SLOW_NCH)])

    for m_hbm, out_hbm in ((m_lo, o_lo), (m_hi, o_hi)):
        def zstore(i, _):
            rows_v[0, i // 4, pl.ds((i % 4) * 16, 16)] = jnp.zeros(
                (16,), jnp.float32)
            return 0

        lax.fori_loop(0, CB * 4, zstore, 0)
        for t in range(RPT // CB):
            pltpu.sync_copy(rows_v.at[0],
                            acc_s.at[pl.ds(sid * RPT + t * CB, CB)])
        plsc.subcore_barrier()

        pltpu.async_copy(m_hbm.at[src_v.at[0]], rows_v.at[0], gsem.at[0])
        pltpu.async_copy(m_hbm.at[src_v.at[1]], rows_v.at[1], gsem.at[1])

        def body(j, _, m_hbm=m_hbm):
            b = lax.rem(j, 4)
            pltpu.make_async_copy(m_hbm.at[src_v.at[j]], rows_v.at[b],
                                  gsem.at[b]).wait()

            @pl.when(j >= 2)
            def _():
                b2 = lax.rem(j - 2, 4)
                pltpu.make_async_copy(rows_v.at[b2],
                                      acc_s.at[dst_v.at[j - 2]],
                                      ssem.at[b2]).wait()

            @pl.when(j + 2 < nch)
            def _():
                b3 = lax.rem(j + 2, 4)
                pltpu.async_copy(m_hbm.at[src_v.at[j + 2]], rows_v.at[b3],
                                 gsem.at[b3])

            pltpu.async_copy(rows_v.at[b], acc_s.at[dst_v.at[j]],
                             ssem.at[b], add=True)
            return 0

        lax.fori_loop(0, nch, body, 0)
        for k in (2, 1):
            jj = nch - k
            b = lax.rem(jj, 4)
            pltpu.make_async_copy(rows_v.at[b], acc_s.at[dst_v.at[jj]],
                                  ssem.at[b]).wait()
        plsc.subcore_barrier()
        pltpu.sync_copy(acc_s.at[pl.ds(sid * RPT, RPT)],
                        out_hbm.at[cid, pl.ds(sid * RPT, RPT)])
        plsc.subcore_barrier()


def _dense1_body(degp, x, w1, lo_o, hi_o, dsc_o):
    deg = jnp.maximum(jnp.sum(degp[...], axis=0), 1.0)
    dsc = lax.rsqrt(deg)
    m = jnp.dot(x[...], w1[...],
                preferred_element_type=jnp.float32) * dsc[:, None]
    lo_o[...] = m[:, :HF]
    hi_o[...] = m[:, HF:]
    dsc_o[...] = dsc


def _dense2_body(alo, ahi, dsc, w2, lo_o, hi_o):
    a = jnp.concatenate([alo[0] + alo[1], ahi[0] + ahi[1]], axis=1)
    h = jnp.maximum(a * dsc[...][:, None], 0.0)
    m = jnp.dot(h, w2[...],
                preferred_element_type=jnp.float32) * dsc[...][:, None]
    lo_o[...] = m[:, :HF]
    hi_o[...] = m[:, HF:]


def _dense3_body(alo, ahi, dsc, eps, wo, bo, z_o):
    t = jnp.concatenate([alo[0] + alo[1], ahi[0] + ahi[1]],
                        axis=1) * dsc[...][:, None]
    mu = jnp.concatenate([t[:, 0:LAT], t[:, 2 * LAT:3 * LAT]], axis=1)
    ls = jnp.concatenate([t[:, LAT:2 * LAT], t[:, 3 * LAT:4 * LAT]], axis=1)
    z = mu + eps[...] * jnp.exp(jnp.minimum(ls, MAXLS))
    z_o[...] = jnp.dot(z, wo[...],
                       preferred_element_type=jnp.float32) + bo[...]


_dense1 = pl.pallas_call(
    _dense1_body,
    grid=(GRID,),
    in_specs=[
        pl.BlockSpec((NW, BN), lambda i: (0, i)),
        pl.BlockSpec((BN, DF), lambda i: (i, 0)),
        pl.BlockSpec((DF, DF), lambda i: (0, 0)),
    ],
    out_specs=[
        pl.BlockSpec((BN, HF), lambda i: (i, 0)),
        pl.BlockSpec((BN, HF), lambda i: (i, 0)),
        pl.BlockSpec((BN,), lambda i: (i,)),
    ],
    out_shape=[
        jax.ShapeDtypeStruct((N, HF), jnp.float32),
        jax.ShapeDtypeStruct((N, HF), jnp.float32),
        jax.ShapeDtypeStruct((N,), jnp.float32),
    ],
)

_agg_spec = pl.BlockSpec((NC, BN, HF), lambda i: (0, i, 0))

_dense2 = pl.pallas_call(
    _dense2_body,
    grid=(GRID,),
    in_specs=[
        _agg_spec,
        _agg_spec,
        pl.BlockSpec((BN,), lambda i: (i,)),
        pl.BlockSpec((DF, DF), lambda i: (0, 0)),
    ],
    out_specs=[
        pl.BlockSpec((BN, HF), lambda i: (i, 0)),
        pl.BlockSpec((BN, HF), lambda i: (i, 0)),
    ],
    out_shape=[
        jax.ShapeDtypeStruct((N, HF), jnp.float32),
        jax.ShapeDtypeStruct((N, HF), jnp.float32),
    ],
)

_dense3 = pl.pallas_call(
    _dense3_body,
    grid=(GRID,),
    in_specs=[
        _agg_spec,
        _agg_spec,
        pl.BlockSpec((BN,), lambda i: (i,)),
        pl.BlockSpec((BN, 2 * LAT), lambda i: (i, 0)),
        pl.BlockSpec((2 * LAT, OUTD), lambda i: (0, 0)),
        pl.BlockSpec((1, OUTD), lambda i: (0, 0)),
    ],
    out_specs=pl.BlockSpec((BN, OUTD), lambda i: (i, 0)),
    out_shape=jax.ShapeDtypeStruct((N, OUTD), jnp.float32),
)


def kernel(x, W1_a, Wmu_a, Wls_a, W1_b, Wmu_b, Wls_b, W_out, b_out,
           edge_index):
    f32 = jnp.float32
    W1c = jnp.concatenate([W1_a, W1_b], axis=1)
    zblk = jnp.zeros((HID, 2 * LAT), f32)
    W2 = jnp.concatenate([
        jnp.concatenate([Wmu_a, Wls_a, zblk], axis=1),
        jnp.concatenate([zblk, Wmu_b, Wls_b], axis=1),
    ], axis=0)
    ke_a, ke_b = jax.random.split(jax.random.key(42), 2)
    eps = jnp.concatenate([
        jax.random.normal(ke_a, (N, LAT), dtype=f32),
        jax.random.normal(ke_b, (N, LAT), dtype=f32),
    ], axis=1)

    pad_n = E_PAD - E
    src_p = jnp.concatenate([edge_index[0],
                             jnp.zeros((pad_n,), jnp.int32)])
    dst_p = jnp.concatenate([edge_index[1],
                             jnp.full((pad_n,), PAD_DST, jnp.int32)])
    dst_f = dst_p.reshape(NW, EPW)

    nfast = FAST_NCH * CB
    spair = edge_index[0].reshape(16, E16)
    dpair = edge_index[1].reshape(16, E16)
    pad_s = SLOW_NCH * CB - (E16 - nfast)
    srcF = spair[:, :nfast].reshape(16, FAST_NCH, CB)
    dstF = dpair[:, :nfast].reshape(16, FAST_NCH, CB)
    srcS = jnp.concatenate(
        [spair[:, nfast:], jnp.zeros((16, pad_s), jnp.int32)],
        axis=1).reshape(16, SLOW_NCH, CB)
    dstS = jnp.concatenate(
        [dpair[:, nfast:], jnp.full((16, pad_s), PAD_DST, jnp.int32)],
        axis=1).reshape(16, SLOW_NCH, CB)

    degp = _sc_degree(dst_f)
    m1lo, m1hi, dsc = _dense1(degp, x, W1c)
    a1lo, a1hi = _sc_agg(m1lo, m1hi, srcF, dstF, srcS, dstS)
    m2lo, m2hi = _dense2(a1lo, a1hi, dsc, W2)
    a2lo, a2hi = _sc_agg(m2lo, m2hi, srcF, dstF, srcS, dstS)
    z = _dense3(a2lo, a2hi, dsc, eps, W_out, b_out.reshape(1, OUTD))
    return z


# speedup vs baseline: 23.7869x; 1.0494x over previous
import functools

import jax
import jax.numpy as jnp
from jax import lax
from jax.experimental import pallas as pl
from jax.experimental.pallas import tpu as pltpu
from jax.experimental.pallas import tpu_sc as plsc

N = 10000
E = 320000
DF = 128
HF = 64
HID = 64
LAT = 32
OUTD = 64
MAXLS = 10.0

NC, NS = 2, 16
NW = NC * NS
EPW = 10240
E_PAD = NW * EPW
CB = 128
NCH = EPW // CB
RPT = 640
R_ACC = NS * RPT
PAD_DST = 10008
DEG_ACC = 10240

BN = 1024
GRID = (N + BN - 1) // BN


def _sc_mesh():
    return plsc.VectorSubcoreMesh(core_axis_name="c", subcore_axis_name="s")


@functools.partial(
    pl.kernel,
    mesh=_sc_mesh(),
    out_type=jax.ShapeDtypeStruct((NW, DEG_ACC), jnp.float32),
    scratch_types=[
        pltpu.VMEM((EPW,), jnp.int32),
        pltpu.VMEM((DEG_ACC,), jnp.float32),
    ],
    compiler_params=pltpu.CompilerParams(needs_layout_passes=False),
)
def _sc_degree(dst_hbm, out_hbm, dst_v, acc_v):
    cid = lax.axis_index("c")
    sid = lax.axis_index("s")
    wid = cid * NS + sid
    pltpu.sync_copy(dst_hbm.at[wid], dst_v)

    def zero(i, _):
        acc_v[pl.ds(i * 16, 16)] = jnp.zeros((16,), jnp.float32)
        return 0

    lax.fori_loop(0, DEG_ACC // 16, zero, 0)

    ones = jnp.ones((16,), jnp.float32)

    def body(i, _):
        idx = dst_v[pl.ds(i * 16, 16)]
        plsc.addupdate_scatter(acc_v, [idx], ones)
        return 0

    lax.fori_loop(0, EPW // 16, body, 0)
    pltpu.sync_copy(acc_v, out_hbm.at[wid])


FAST_CID = 0
FAST_NCH = 109
SLOW_NCH = 48
E16 = E // 16


@functools.partial(
    pl.kernel,
    mesh=_sc_mesh(),
    out_type=[
        jax.ShapeDtypeStruct((NC, R_ACC, HF), jnp.float32),
        jax.ShapeDtypeStruct((NC, R_ACC, HF), jnp.float32),
    ],
    scratch_types=[
        pltpu.VMEM((FAST_NCH, CB), jnp.int32),
        pltpu.VMEM((FAST_NCH, CB), jnp.int32),
        pltpu.VMEM((4, CB, HF), jnp.float32),
        pltpu.VMEM_SHARED((R_ACC, HF), jnp.float32),
        pltpu.SemaphoreType.DMA((4,)),
        pltpu.SemaphoreType.DMA((4,)),
    ],
    compiler_params=pltpu.CompilerParams(use_tc_tiling_on_sc=False),
)
def _sc_agg(m_lo, m_hi, srcF, dstF, srcS, dstS, o_lo, o_hi,
            src_v, dst_v, rows_v, acc_s, gsem, ssem):
    cid = lax.axis_index("c")
    sid = lax.axis_index("s")
    is_fast = cid == FAST_CID
    nch = jnp.where(is_fast, FAST_NCH, SLOW_NCH)

    @pl.when(is_fast)
    def _():
        pltpu.sync_copy(srcF.at[sid], src_v)
        pltpu.sync_copy(dstF.at[sid], dst_v)

    @pl.when(jnp.logical_not(is_fast))
    def _():
        pltpu.sync_copy(srcS.at[sid], src_v.at[pl.ds(0, SLOW_NCH)])
        pltpu.sync_copy(dstS.at[sid], dst_v.at[pl.ds(0, SLOW_NCH)])

    for m_hbm, out_hbm in ((m_lo, o_lo), (m_hi, o_hi)):
        def zstore(i, _):
            rows_v[0, i // 4, pl.ds((i % 4) * 16, 16)] = jnp.zeros(
                (16,), jnp.float32)
            return 0

        lax.fori_loop(0, CB * 4, zstore, 0)
        for t in range(RPT // CB):
            pltpu.sync_copy(rows_v.at[0],
                            acc_s.at[pl.ds(sid * RPT + t * CB, CB)])
        plsc.subcore_barrier()

        pltpu.async_copy(m_hbm.at[src_v.at[0]], rows_v.at[0], gsem.at[0])
        pltpu.async_copy(m_hbm.at[src_v.at[1]], rows_v.at[1], gsem.at[1])

        def body(j, _, m_hbm=m_hbm):
            b = lax.rem(j, 4)
            pltpu.make_async_copy(m_hbm.at[src_v.at[j]], rows_v.at[b],
                                  gsem.at[b]).wait()

            @pl.when(j >= 2)
            def _():
                b2 = lax.rem(j - 2, 4)
                pltpu.make_async_copy(rows_v.at[b2],
                                      acc_s.at[dst_v.at[j - 2]],
                                      ssem.at[b2]).wait()

            @pl.when(j + 2 < nch)
            def _():
                b3 = lax.rem(j + 2, 4)
                pltpu.async_copy(m_hbm.at[src_v.at[j + 2]], rows_v.at[b3],
                                 gsem.at[b3])

            pltpu.async_copy(rows_v.at[b], acc_s.at[dst_v.at[j]],
                             ssem.at[b], add=True)
            return 0

        lax.fori_loop(0, nch, body, 0)
        for k in (2, 1):
            jj = nch - k
            b = lax.rem(jj, 4)
            pltpu.make_async_copy(rows_v.at[b], acc_s.at[dst_v.at[jj]],
                                  ssem.at[b]).wait()
        plsc.subcore_barrier()
        pltpu.sync_copy(acc_s.at[pl.ds(sid * RPT, RPT)],
                        out_hbm.at[cid, pl.ds(sid * RPT, RPT)])
        plsc.subcore_barrier()


def _dense1_body(degp, x, w1, lo_o, hi_o, dsc_o):
    deg = jnp.maximum(jnp.sum(degp[...], axis=0), 1.0)
    dsc = lax.rsqrt(deg)
    m = jnp.dot(x[...], w1[...],
                preferred_element_type=jnp.float32) * dsc[:, None]
    lo_o[...] = m[:, :HF]
    hi_o[...] = m[:, HF:]
    dsc_o[...] = dsc


def _dense2_body(alo, ahi, dsc, w2, lo_o, hi_o):
    a = jnp.concatenate([alo[0] + alo[1], ahi[0] + ahi[1]], axis=1)
    h = jnp.maximum(a * dsc[...][:, None], 0.0)
    m = jnp.dot(h, w2[...],
                preferred_element_type=jnp.float32) * dsc[...][:, None]
    lo_o[...] = m[:, :HF]
    hi_o[...] = m[:, HF:]


def _dense3_body(alo, ahi, dsc, eps, wo, bo, z_o):
    t = jnp.concatenate([alo[0] + alo[1], ahi[0] + ahi[1]],
                        axis=1) * dsc[...][:, None]
    mu = jnp.concatenate([t[:, 0:LAT], t[:, 2 * LAT:3 * LAT]], axis=1)
    ls = jnp.concatenate([t[:, LAT:2 * LAT], t[:, 3 * LAT:4 * LAT]], axis=1)
    z = mu + eps[...] * jnp.exp(jnp.minimum(ls, MAXLS))
    z_o[...] = jnp.dot(z, wo[...],
                       preferred_element_type=jnp.float32) + bo[...]


_dense1 = pl.pallas_call(
    _dense1_body,
    grid=(GRID,),
    in_specs=[
        pl.BlockSpec((NW, BN), lambda i: (0, i)),
        pl.BlockSpec((BN, DF), lambda i: (i, 0)),
        pl.BlockSpec((DF, DF), lambda i: (0, 0)),
    ],
    out_specs=[
        pl.BlockSpec((BN, HF), lambda i: (i, 0)),
        pl.BlockSpec((BN, HF), lambda i: (i, 0)),
        pl.BlockSpec((BN,), lambda i: (i,)),
    ],
    out_shape=[
        jax.ShapeDtypeStruct((N, HF), jnp.float32),
        jax.ShapeDtypeStruct((N, HF), jnp.float32),
        jax.ShapeDtypeStruct((N,), jnp.float32),
    ],
)

_agg_spec = pl.BlockSpec((NC, BN, HF), lambda i: (0, i, 0))

_dense2 = pl.pallas_call(
    _dense2_body,
    grid=(GRID,),
    in_specs=[
        _agg_spec,
        _agg_spec,
        pl.BlockSpec((BN,), lambda i: (i,)),
        pl.BlockSpec((DF, DF), lambda i: (0, 0)),
    ],
    out_specs=[
        pl.BlockSpec((BN, HF), lambda i: (i, 0)),
        pl.BlockSpec((BN, HF), lambda i: (i, 0)),
    ],
    out_shape=[
        jax.ShapeDtypeStruct((N, HF), jnp.float32),
        jax.ShapeDtypeStruct((N, HF), jnp.float32),
    ],
)

_dense3 = pl.pallas_call(
    _dense3_body,
    grid=(GRID,),
    in_specs=[
        _agg_spec,
        _agg_spec,
        pl.BlockSpec((BN,), lambda i: (i,)),
        pl.BlockSpec((BN, 2 * LAT), lambda i: (i, 0)),
        pl.BlockSpec((2 * LAT, OUTD), lambda i: (0, 0)),
        pl.BlockSpec((1, OUTD), lambda i: (0, 0)),
    ],
    out_specs=pl.BlockSpec((BN, OUTD), lambda i: (i, 0)),
    out_shape=jax.ShapeDtypeStruct((N, OUTD), jnp.float32),
)


def kernel(x, W1_a, Wmu_a, Wls_a, W1_b, Wmu_b, Wls_b, W_out, b_out,
           edge_index):
    f32 = jnp.float32
    W1c = jnp.concatenate([W1_a, W1_b], axis=1)
    zblk = jnp.zeros((HID, 2 * LAT), f32)
    W2 = jnp.concatenate([
        jnp.concatenate([Wmu_a, Wls_a, zblk], axis=1),
        jnp.concatenate([zblk, Wmu_b, Wls_b], axis=1),
    ], axis=0)
    ke_a, ke_b = jax.random.split(jax.random.key(42), 2)
    eps = jnp.concatenate([
        jax.random.normal(ke_a, (N, LAT), dtype=f32),
        jax.random.normal(ke_b, (N, LAT), dtype=f32),
    ], axis=1)

    pad_n = E_PAD - E
    src_p = jnp.concatenate([edge_index[0],
                             jnp.zeros((pad_n,), jnp.int32)])
    dst_p = jnp.concatenate([edge_index[1],
                             jnp.full((pad_n,), PAD_DST, jnp.int32)])
    dst_f = dst_p.reshape(NW, EPW)

    nfast = FAST_NCH * CB
    spair = edge_index[0].reshape(16, E16)
    dpair = edge_index[1].reshape(16, E16)
    pad_s = SLOW_NCH * CB - (E16 - nfast)
    srcF = spair[:, :nfast].reshape(16, FAST_NCH, CB)
    dstF = dpair[:, :nfast].reshape(16, FAST_NCH, CB)
    srcS = jnp.concatenate(
        [spair[:, nfast:], jnp.zeros((16, pad_s), jnp.int32)],
        axis=1).reshape(16, SLOW_NCH, CB)
    dstS = jnp.concatenate(
        [dpair[:, nfast:], jnp.full((16, pad_s), PAD_DST, jnp.int32)],
        axis=1).reshape(16, SLOW_NCH, CB)

    degp = _sc_degree(dst_f)
    m1lo, m1hi, dsc = _dense1(degp, x, W1c)
    a1lo, a1hi = _sc_agg(m1lo, m1hi, srcF, dstF, srcS, dstS)
    m2lo, m2hi = _dense2(a1lo, a1hi, dsc, W2)
    a2lo, a2hi = _sc_agg(m2lo, m2hi, srcF, dstF, srcS, dstS)
    z = _dense3(a2lo, a2hi, dsc, eps, W_out, b_out.reshape(1, OUTD))
    return z


# speedup vs baseline: 26.2839x; 1.1050x over previous
import functools

import jax
import jax.numpy as jnp
from jax import lax
from jax.experimental import pallas as pl
from jax.experimental.pallas import tpu as pltpu
from jax.experimental.pallas import tpu_sc as plsc

N = 10000
E = 320000
DF = 128
HF = 64
HID = 64
LAT = 32
OUTD = 64
MAXLS = 10.0

NC, NS = 2, 16
NW = NC * NS
EPW = 10240
E_PAD = NW * EPW
CB = 128
NCH = EPW // CB
RPT = 640
R_ACC = NS * RPT
PAD_DST = 10008
DEG_ACC = 10240

BN = 1024
GRID = (N + BN - 1) // BN


def _sc_mesh():
    return plsc.VectorSubcoreMesh(core_axis_name="c", subcore_axis_name="s")


EPD = E // NW


@functools.partial(
    pl.kernel,
    mesh=_sc_mesh(),
    out_type=jax.ShapeDtypeStruct((NW, DEG_ACC), jnp.float32),
    scratch_types=[
        pltpu.VMEM((EPD,), jnp.int32),
        pltpu.VMEM((DEG_ACC,), jnp.float32),
    ],
    compiler_params=pltpu.CompilerParams(needs_layout_passes=False),
)
def _sc_degree(dst_hbm, out_hbm, dst_v, acc_v):
    cid = lax.axis_index("c")
    sid = lax.axis_index("s")
    wid = cid * NS + sid
    pltpu.sync_copy(dst_hbm.at[pl.ds(wid * EPD, EPD)], dst_v)

    def zero(i, _):
        acc_v[pl.ds(i * 16, 16)] = jnp.zeros((16,), jnp.float32)
        return 0

    lax.fori_loop(0, DEG_ACC // 16, zero, 0)

    ones = jnp.ones((16,), jnp.float32)

    def body(i, _):
        idx = dst_v[pl.ds(i * 16, 16)]
        plsc.addupdate_scatter(acc_v, [idx], ones)
        return 0

    lax.fori_loop(0, EPD // 16, body, 0)
    pltpu.sync_copy(acc_v, out_hbm.at[wid])


FAST_CID = 0
FAST_NCH = 109
SLOW_NCH = 48
ECH = E // CB
FROWS = 16 * FAST_NCH
SROWS = 16 * SLOW_NCH


@functools.partial(
    pl.kernel,
    mesh=_sc_mesh(),
    out_type=[
        jax.ShapeDtypeStruct((NC, R_ACC, HF), jnp.float32),
        jax.ShapeDtypeStruct((NC, R_ACC, HF), jnp.float32),
    ],
    scratch_types=[
        pltpu.VMEM((FAST_NCH, CB), jnp.int32),
        pltpu.VMEM((FAST_NCH, CB), jnp.int32),
        pltpu.VMEM((4, CB, HF), jnp.float32),
        pltpu.VMEM_SHARED((R_ACC, HF), jnp.float32),
        pltpu.SemaphoreType.DMA((4,)),
        pltpu.SemaphoreType.DMA((4,)),
    ],
    compiler_params=pltpu.CompilerParams(use_tc_tiling_on_sc=False),
)
def _sc_agg(m_lo, m_hi, src3, dst3, srcS, dstS, o_lo, o_hi,
            src_v, dst_v, rows_v, acc_s, gsem, ssem):
    cid = lax.axis_index("c")
    sid = lax.axis_index("s")
    is_fast = cid == FAST_CID
    nch = jnp.where(is_fast, FAST_NCH, SLOW_NCH)

    @pl.when(is_fast)
    def _():
        pltpu.sync_copy(src3.at[pl.ds(sid * FAST_NCH, FAST_NCH)], src_v)
        pltpu.sync_copy(dst3.at[pl.ds(sid * FAST_NCH, FAST_NCH)], dst_v)

    @pl.when(jnp.logical_not(is_fast))
    def _():
        pltpu.sync_copy(srcS.at[pl.ds(sid * SLOW_NCH, SLOW_NCH)],
                        src_v.at[pl.ds(0, SLOW_NCH)])
        pltpu.sync_copy(dstS.at[pl.ds(sid * SLOW_NCH, SLOW_NCH)],
                        dst_v.at[pl.ds(0, SLOW_NCH)])

    for m_hbm, out_hbm in ((m_lo, o_lo), (m_hi, o_hi)):
        def zstore(i, _):
            rows_v[0, i // 4, pl.ds((i % 4) * 16, 16)] = jnp.zeros(
                (16,), jnp.float32)
            return 0

        lax.fori_loop(0, CB * 4, zstore, 0)
        for t in range(RPT // CB):
            pltpu.sync_copy(rows_v.at[0],
                            acc_s.at[pl.ds(sid * RPT + t * CB, CB)])
        plsc.subcore_barrier()

        pltpu.async_copy(m_hbm.at[src_v.at[0]], rows_v.at[0], gsem.at[0])
        pltpu.async_copy(m_hbm.at[src_v.at[1]], rows_v.at[1], gsem.at[1])

        def body(j, _, m_hbm=m_hbm):
            b = lax.rem(j, 4)
            pltpu.make_async_copy(m_hbm.at[src_v.at[j]], rows_v.at[b],
                                  gsem.at[b]).wait()

            @pl.when(j >= 2)
            def _():
                b2 = lax.rem(j - 2, 4)
                pltpu.make_async_copy(rows_v.at[b2],
                                      acc_s.at[dst_v.at[j - 2]],
                                      ssem.at[b2]).wait()

            @pl.when(j + 2 < nch)
            def _():
                b3 = lax.rem(j + 2, 4)
                pltpu.async_copy(m_hbm.at[src_v.at[j + 2]], rows_v.at[b3],
                                 gsem.at[b3])

            pltpu.async_copy(rows_v.at[b], acc_s.at[dst_v.at[j]],
                             ssem.at[b], add=True)
            return 0

        lax.fori_loop(0, nch, body, 0)
        for k in (2, 1):
            jj = nch - k
            b = lax.rem(jj, 4)
            pltpu.make_async_copy(rows_v.at[b], acc_s.at[dst_v.at[jj]],
                                  ssem.at[b]).wait()
        plsc.subcore_barrier()
        pltpu.sync_copy(acc_s.at[pl.ds(sid * RPT, RPT)],
                        out_hbm.at[cid, pl.ds(sid * RPT, RPT)])
        plsc.subcore_barrier()


def _dense1a_body(x, w1, u_o):
    u_o[...] = jnp.dot(x[...], w1[...], preferred_element_type=jnp.float32)


def _dense1b_body(degp, u, lo_o, hi_o, dsc_o):
    deg = jnp.maximum(jnp.sum(degp[...], axis=0), 1.0)
    dsc = lax.rsqrt(deg)
    m = u[...] * dsc[:, None]
    lo_o[...] = m[:, :HF]
    hi_o[...] = m[:, HF:]
    dsc_o[...] = dsc


def _dense2_body(alo, ahi, dsc, w2, lo_o, hi_o):
    a = jnp.concatenate([alo[0] + alo[1], ahi[0] + ahi[1]], axis=1)
    h = jnp.maximum(a * dsc[...][:, None], 0.0)
    m = jnp.dot(h, w2[...],
                preferred_element_type=jnp.float32) * dsc[...][:, None]
    lo_o[...] = m[:, :HF]
    hi_o[...] = m[:, HF:]


def _dense3_body(alo, ahi, dsc, eps, wo, bo, z_o):
    t = jnp.concatenate([alo[0] + alo[1], ahi[0] + ahi[1]],
                        axis=1) * dsc[...][:, None]
    mu = jnp.concatenate([t[:, 0:LAT], t[:, 2 * LAT:3 * LAT]], axis=1)
    ls = jnp.concatenate([t[:, LAT:2 * LAT], t[:, 3 * LAT:4 * LAT]], axis=1)
    z = mu + eps[...] * jnp.exp(jnp.minimum(ls, MAXLS))
    z_o[...] = jnp.dot(z, wo[...],
                       preferred_element_type=jnp.float32) + bo[...]


_dense1a = pl.pallas_call(
    _dense1a_body,
    grid=(GRID,),
    in_specs=[
        pl.BlockSpec((BN, DF), lambda i: (i, 0)),
        pl.BlockSpec((DF, DF), lambda i: (0, 0)),
    ],
    out_specs=pl.BlockSpec((BN, DF), lambda i: (i, 0)),
    out_shape=jax.ShapeDtypeStruct((N, DF), jnp.float32),
)

_dense1b = pl.pallas_call(
    _dense1b_body,
    grid=(GRID,),
    in_specs=[
        pl.BlockSpec((NW, BN), lambda i: (0, i)),
        pl.BlockSpec((BN, DF), lambda i: (i, 0)),
    ],
    out_specs=[
        pl.BlockSpec((BN, HF), lambda i: (i, 0)),
        pl.BlockSpec((BN, HF), lambda i: (i, 0)),
        pl.BlockSpec((BN,), lambda i: (i,)),
    ],
    out_shape=[
        jax.ShapeDtypeStruct((N, HF), jnp.float32),
        jax.ShapeDtypeStruct((N, HF), jnp.float32),
        jax.ShapeDtypeStruct((N,), jnp.float32),
    ],
)

_agg_spec = pl.BlockSpec((NC, BN, HF), lambda i: (0, i, 0))

_dense2 = pl.pallas_call(
    _dense2_body,
    grid=(GRID,),
    in_specs=[
        _agg_spec,
        _agg_spec,
        pl.BlockSpec((BN,), lambda i: (i,)),
        pl.BlockSpec((DF, DF), lambda i: (0, 0)),
    ],
    out_specs=[
        pl.BlockSpec((BN, HF), lambda i: (i, 0)),
        pl.BlockSpec((BN, HF), lambda i: (i, 0)),
    ],
    out_shape=[
        jax.ShapeDtypeStruct((N, HF), jnp.float32),
        jax.ShapeDtypeStruct((N, HF), jnp.float32),
    ],
)

_dense3 = pl.pallas_call(
    _dense3_body,
    grid=(GRID,),
    in_specs=[
        _agg_spec,
        _agg_spec,
        pl.BlockSpec((BN,), lambda i: (i,)),
        pl.BlockSpec((BN, 2 * LAT), lambda i: (i, 0)),
        pl.BlockSpec((2 * LAT, OUTD), lambda i: (0, 0)),
        pl.BlockSpec((1, OUTD), lambda i: (0, 0)),
    ],
    out_specs=pl.BlockSpec((BN, OUTD), lambda i: (i, 0)),
    out_shape=jax.ShapeDtypeStruct((N, OUTD), jnp.float32),
)


def kernel(x, W1_a, Wmu_a, Wls_a, W1_b, Wmu_b, Wls_b, W_out, b_out,
           edge_index):
    f32 = jnp.float32
    W1c = jnp.concatenate([W1_a, W1_b], axis=1)
    zblk = jnp.zeros((HID, 2 * LAT), f32)
    W2 = jnp.concatenate([
        jnp.concatenate([Wmu_a, Wls_a, zblk], axis=1),
        jnp.concatenate([zblk, Wmu_b, Wls_b], axis=1),
    ], axis=0)
    ke_a, ke_b = jax.random.split(jax.random.key(42), 2)
    eps = jnp.concatenate([
        jax.random.normal(ke_a, (N, LAT), dtype=f32),
        jax.random.normal(ke_b, (N, LAT), dtype=f32),
    ], axis=1)

    src3 = edge_index[0].reshape(ECH, CB)
    dst3 = edge_index[1].reshape(ECH, CB)
    npad = SROWS - (ECH - FROWS)
    srcS = jnp.concatenate(
        [src3[FROWS:], jnp.zeros((npad, CB), jnp.int32)])
    dstS = jnp.concatenate(
        [dst3[FROWS:], jnp.full((npad, CB), PAD_DST, jnp.int32)])

    degp = _sc_degree(edge_index[1])
    u = _dense1a(x, W1c)
    m1lo, m1hi, dsc = _dense1b(degp, u)
    a1lo, a1hi = _sc_agg(m1lo, m1hi, src3, dst3, srcS, dstS)
    m2lo, m2hi = _dense2(a1lo, a1hi, dsc, W2)
    a2lo, a2hi = _sc_agg(m2lo, m2hi, src3, dst3, srcS, dstS)
    z = _dense3(a2lo, a2hi, dsc, eps, W_out, b_out.reshape(1, OUTD))
    return z


# speedup vs baseline: 28.6295x; 1.0892x over previous
import functools

import jax
import jax.numpy as jnp
from jax import lax
from jax.experimental import pallas as pl
from jax.experimental.pallas import tpu as pltpu
from jax.experimental.pallas import tpu_sc as plsc

N = 10000
E = 320000
DF = 128
HF = 64
HID = 64
LAT = 32
OUTD = 64
MAXLS = 10.0

NC, NS = 2, 16
NW = NC * NS
EPW = 10240
E_PAD = NW * EPW
CB = 128
NCH = EPW // CB
RPT = 640
R_ACC = NS * RPT
PAD_DST = 10008
DEG_ACC = 10240

BN = 1024
GRID = (N + BN - 1) // BN


def _sc_mesh():
    return plsc.VectorSubcoreMesh(core_axis_name="c", subcore_axis_name="s")


EPD = E // NW


@functools.partial(
    pl.kernel,
    mesh=_sc_mesh(),
    out_type=jax.ShapeDtypeStruct((NW, DEG_ACC), jnp.float32),
    scratch_types=[
        pltpu.VMEM((EPD,), jnp.int32),
        pltpu.VMEM((DEG_ACC,), jnp.float32),
    ],
    compiler_params=pltpu.CompilerParams(needs_layout_passes=False),
)
def _sc_degree(dst_hbm, out_hbm, dst_v, acc_v):
    cid = lax.axis_index("c")
    sid = lax.axis_index("s")
    wid = cid * NS + sid
    pltpu.sync_copy(dst_hbm.at[pl.ds(wid * EPD, EPD)], dst_v)

    def zero(i, _):
        acc_v[pl.ds(i * 16, 16)] = jnp.zeros((16,), jnp.float32)
        return 0

    lax.fori_loop(0, DEG_ACC // 16, zero, 0)

    ones = jnp.ones((16,), jnp.float32)

    def body(i, _):
        idx = dst_v[pl.ds(i * 16, 16)]
        plsc.addupdate_scatter(acc_v, [idx], ones)
        return 0

    lax.fori_loop(0, EPD // 16, body, 0)
    pltpu.sync_copy(acc_v, out_hbm.at[wid])


FAST_CID = 0
FAST_NCH = 106
SLOW_NCH = 51
ECH = E // CB
FROWS = 16 * FAST_NCH
SROWS = 16 * SLOW_NCH


@functools.partial(
    pl.kernel,
    mesh=_sc_mesh(),
    out_type=jax.ShapeDtypeStruct((NC, R_ACC, HF), jnp.float32),
    scratch_types=[
        pltpu.VMEM((FAST_NCH, CB), jnp.int32),
        pltpu.VMEM((FAST_NCH, CB), jnp.int32),
        pltpu.VMEM((4, CB, HF), jnp.float32),
        pltpu.VMEM_SHARED((R_ACC, HF), jnp.float32),
        pltpu.SemaphoreType.DMA((4,)),
        pltpu.SemaphoreType.DMA((4,)),
    ],
    compiler_params=pltpu.CompilerParams(use_tc_tiling_on_sc=False),
)
def _sc_agg(m_hbm, src3, dst3, srcS, dstS, out_hbm,
            src_v, dst_v, rows_v, acc_s, gsem, ssem):
    cid = lax.axis_index("c")
    sid = lax.axis_index("s")
    is_fast = cid == FAST_CID
    nch = jnp.where(is_fast, FAST_NCH, SLOW_NCH)

    @pl.when(is_fast)
    def _():
        pltpu.sync_copy(src3.at[pl.ds(sid * FAST_NCH, FAST_NCH)], src_v)
        pltpu.sync_copy(dst3.at[pl.ds(sid * FAST_NCH, FAST_NCH)], dst_v)

    @pl.when(jnp.logical_not(is_fast))
    def _():
        pltpu.sync_copy(srcS.at[pl.ds(sid * SLOW_NCH, SLOW_NCH)],
                        src_v.at[pl.ds(0, SLOW_NCH)])
        pltpu.sync_copy(dstS.at[pl.ds(sid * SLOW_NCH, SLOW_NCH)],
                        dst_v.at[pl.ds(0, SLOW_NCH)])

    def zstore(i, _):
        rows_v[0, i // 4, pl.ds((i % 4) * 16, 16)] = jnp.zeros(
            (16,), jnp.float32)
        return 0

    lax.fori_loop(0, CB * 4, zstore, 0)
    for t in range(RPT // CB):
        pltpu.sync_copy(rows_v.at[0],
                        acc_s.at[pl.ds(sid * RPT + t * CB, CB)])
    plsc.subcore_barrier()

    pltpu.async_copy(m_hbm.at[src_v.at[0]], rows_v.at[0], gsem.at[0])
    pltpu.async_copy(m_hbm.at[src_v.at[1]], rows_v.at[1], gsem.at[1])

    def body(j, _):
        b = lax.rem(j, 4)
        pltpu.make_async_copy(m_hbm.at[src_v.at[j]], rows_v.at[b],
                              gsem.at[b]).wait()

        @pl.when(j >= 2)
        def _():
            b2 = lax.rem(j - 2, 4)
            pltpu.make_async_copy(rows_v.at[b2],
                                  acc_s.at[dst_v.at[j - 2]],
                                  ssem.at[b2]).wait()

        @pl.when(j + 2 < nch)
        def _():
            b3 = lax.rem(j + 2, 4)
            pltpu.async_copy(m_hbm.at[src_v.at[j + 2]], rows_v.at[b3],
                             gsem.at[b3])

        pltpu.async_copy(rows_v.at[b], acc_s.at[dst_v.at[j]],
                         ssem.at[b], add=True)
        return 0

    lax.fori_loop(0, nch, body, 0)
    for k in (2, 1):
        jj = nch - k
        b = lax.rem(jj, 4)
        pltpu.make_async_copy(rows_v.at[b], acc_s.at[dst_v.at[jj]],
                              ssem.at[b]).wait()
    plsc.subcore_barrier()
    pltpu.sync_copy(acc_s.at[pl.ds(sid * RPT, RPT)],
                    out_hbm.at[cid, pl.ds(sid * RPT, RPT)])


def _dense1a_body(x, w1, u_o):
    u_o[...] = jnp.dot(x[...], w1[...], preferred_element_type=jnp.float32)


def _dense1b_body(degp, u, lo_o, hi_o, dsc_o):
    deg = jnp.maximum(jnp.sum(degp[...], axis=0), 1.0)
    dsc = lax.rsqrt(deg)
    m = u[...] * dsc[:, None]
    lo_o[...] = m[:, :HF]
    hi_o[...] = m[:, HF:]
    dsc_o[...] = dsc


def _dense2h_body(a, dsc, w, m_o):
    h = jnp.maximum((a[0] + a[1]) * dsc[...][:, None], 0.0)
    m_o[...] = jnp.dot(h, w[...],
                       preferred_element_type=jnp.float32) * dsc[...][:, None]


def _dense3_body(alo, ahi, dsc, eps, wo, bo, z_o):
    t = jnp.concatenate([alo[0] + alo[1], ahi[0] + ahi[1]],
                        axis=1) * dsc[...][:, None]
    mu = jnp.concatenate([t[:, 0:LAT], t[:, 2 * LAT:3 * LAT]], axis=1)
    ls = jnp.concatenate([t[:, LAT:2 * LAT], t[:, 3 * LAT:4 * LAT]], axis=1)
    z = mu + eps[...] * jnp.exp(jnp.minimum(ls, MAXLS))
    z_o[...] = jnp.dot(z, wo[...],
                       preferred_element_type=jnp.float32) + bo[...]


_dense1a = pl.pallas_call(
    _dense1a_body,
    grid=(GRID,),
    in_specs=[
        pl.BlockSpec((BN, DF), lambda i: (i, 0)),
        pl.BlockSpec((DF, DF), lambda i: (0, 0)),
    ],
    out_specs=pl.BlockSpec((BN, DF), lambda i: (i, 0)),
    out_shape=jax.ShapeDtypeStruct((N, DF), jnp.float32),
)

_dense1b = pl.pallas_call(
    _dense1b_body,
    grid=(GRID,),
    in_specs=[
        pl.BlockSpec((NW, BN), lambda i: (0, i)),
        pl.BlockSpec((BN, DF), lambda i: (i, 0)),
    ],
    out_specs=[
        pl.BlockSpec((BN, HF), lambda i: (i, 0)),
        pl.BlockSpec((BN, HF), lambda i: (i, 0)),
        pl.BlockSpec((BN,), lambda i: (i,)),
    ],
    out_shape=[
        jax.ShapeDtypeStruct((N, HF), jnp.float32),
        jax.ShapeDtypeStruct((N, HF), jnp.float32),
        jax.ShapeDtypeStruct((N,), jnp.float32),
    ],
)

_agg_spec = pl.BlockSpec((NC, BN, HF), lambda i: (0, i, 0))

_dense2h = pl.pallas_call(
    _dense2h_body,
    grid=(GRID,),
    in_specs=[
        _agg_spec,
        pl.BlockSpec((BN,), lambda i: (i,)),
        pl.BlockSpec((HF, HF), lambda i: (0, 0)),
    ],
    out_specs=pl.BlockSpec((BN, HF), lambda i: (i, 0)),
    out_shape=jax.ShapeDtypeStruct((N, HF), jnp.float32),
)

_dense3 = pl.pallas_call(
    _dense3_body,
    grid=(GRID,),
    in_specs=[
        _agg_spec,
        _agg_spec,
        pl.BlockSpec((BN,), lambda i: (i,)),
        pl.BlockSpec((BN, 2 * LAT), lambda i: (i, 0)),
        pl.BlockSpec((2 * LAT, OUTD), lambda i: (0, 0)),
        pl.BlockSpec((1, OUTD), lambda i: (0, 0)),
    ],
    out_specs=pl.BlockSpec((BN, OUTD), lambda i: (i, 0)),
    out_shape=jax.ShapeDtypeStruct((N, OUTD), jnp.float32),
)


def kernel(x, W1_a, Wmu_a, Wls_a, W1_b, Wmu_b, Wls_b, W_out, b_out,
           edge_index):
    f32 = jnp.float32
    W1c = jnp.concatenate([W1_a, W1_b], axis=1)
    W2A = jnp.concatenate([Wmu_a, Wls_a], axis=1)
    W2B = jnp.concatenate([Wmu_b, Wls_b], axis=1)
    ke_a, ke_b = jax.random.split(jax.random.key(42), 2)
    eps = jnp.concatenate([
        jax.random.normal(ke_a, (N, LAT), dtype=f32),
        jax.random.normal(ke_b, (N, LAT), dtype=f32),
    ], axis=1)

    src3 = edge_index[0].reshape(ECH, CB)
    dst3 = edge_index[1].reshape(ECH, CB)
    npad = SROWS - (ECH - FROWS)
    srcS = jnp.concatenate(
        [src3[FROWS:], jnp.zeros((npad, CB), jnp.int32)])
    dstS = jnp.concatenate(
        [dst3[FROWS:], jnp.full((npad, CB), PAD_DST, jnp.int32)])

    degp = _sc_degree(edge_index[1])
    u = _dense1a(x, W1c)
    m1lo, m1hi, dsc = _dense1b(degp, u)
    a1lo = _sc_agg(m1lo, src3, dst3, srcS, dstS)
    m2lo = _dense2h(a1lo, dsc, W2A)
    a1hi = _sc_agg(m1hi, src3, dst3, srcS, dstS)
    m2hi = _dense2h(a1hi, dsc, W2B)
    a2lo = _sc_agg(m2lo, src3, dst3, srcS, dstS)
    a2hi = _sc_agg(m2hi, src3, dst3, srcS, dstS)
    z = _dense3(a2lo, a2hi, dsc, eps, W_out, b_out.reshape(1, OUTD))
    return z
